# Initial kernel scaffold; baseline (speedup 1.0000x reference)
import jax
import jax.numpy as jnp
from jax.experimental import pallas as pl


def kernel(x_ped, x_neighbor, x_sign, cls_neighbor, cls_sign, seg_neighbor, seg_sign, W_ped, b_ped, W_nb, b_nb, W_sg, b_sg, W1, b1, W2, b2, W3, b3):
    raise NotImplementedError("write your pallas kernel here")


# speedup vs baseline: 4.3301x; 4.3301x over previous
import functools

import jax
import jax.numpy as jnp
import numpy as np
from jax.experimental import pallas as pl

_B = 16
_T = 16
_N = 32768
_BN = 2048


def _body(xn_ref, cn_ref, segn_ref, xs_ref, cs_ref, segs_ref, xped_ref,
          BDn_ref, bn_ref, BDs_ref, bs_ref, E_ref,
          BDp_ref, bp_ref, BD1p_ref, BD1n_ref, BD1s_ref, b1_ref,
          BD2_ref, b2_ref, BD3_ref, b3_ref,
          Fnb_ref, Fsg_ref, P_ref, S_ref):
    i = pl.program_id(0)

    def agg(x_ref, c_ref, seg_ref, BD_ref, b_ref):
        e = jnp.maximum(
            jnp.dot(x_ref[...], BD_ref[...], preferred_element_type=jnp.float32)
            + b_ref[...], 0.0)
        maskf = (c_ref[...] != -1.0).astype(jnp.float32)
        m = jnp.dot(maskf, E_ref[...], preferred_element_type=jnp.float32)
        em = e * m
        seg = seg_ref[0]
        ohT = (jax.lax.broadcasted_iota(jnp.int32, (_B, _BN), 0)
               == jnp.broadcast_to(seg, (_B, _BN))).astype(jnp.float32)
        return jnp.dot(ohT, em, preferred_element_type=jnp.float32)

    pn = agg(xn_ref, cn_ref, segn_ref, BDn_ref, bn_ref)
    ps = agg(xs_ref, cs_ref, segs_ref, BDs_ref, bs_ref)

    @pl.when(i == 0)
    def _():
        Fnb_ref[...] = pn
        Fsg_ref[...] = ps

    @pl.when(i > 0)
    def _():
        Fnb_ref[...] += pn
        Fsg_ref[...] += ps

    @pl.when(i == pl.num_programs(0) - 1)
    def _():
        P = jnp.maximum(
            jnp.dot(xped_ref[...], BDp_ref[...], preferred_element_type=jnp.float32)
            + bp_ref[...], 0.0)
        P_ref[...] = P
        A1 = jnp.maximum(
            jnp.dot(P, BD1p_ref[...], preferred_element_type=jnp.float32)
            + jnp.dot(Fnb_ref[...], BD1n_ref[...], preferred_element_type=jnp.float32)
            + jnp.dot(Fsg_ref[...], BD1s_ref[...], preferred_element_type=jnp.float32)
            + b1_ref[...], 0.0)
        A2 = jnp.maximum(
            jnp.dot(A1, BD2_ref[...], preferred_element_type=jnp.float32)
            + b2_ref[...], 0.0)
        S_ref[...] = (jnp.dot(A2, BD3_ref[...], preferred_element_type=jnp.float32)
                      + b3_ref[...])


@functools.partial(jax.jit, static_argnames=("interpret",))
def _run(x_ped, x_neighbor, x_sign, cls_neighbor, cls_sign,
         seg_neighbor, seg_sign,
         W_ped, b_ped, W_nb, b_nb, W_sg, b_sg,
         W1, b1, W2, b2, W3, b3, interpret=False):
    NB = _N // _BN
    eyeT = jnp.eye(_T, dtype=jnp.float32)

    def bd(W):
        return jnp.kron(eyeT, W)

    xn2 = x_neighbor.reshape(_N, _T * 4)
    xs2 = x_sign.reshape(_N, _T * 2)
    xped2 = x_ped.reshape(_B, _T * 4)
    segn3 = seg_neighbor.reshape(NB, 1, _BN)
    segs3 = seg_sign.reshape(NB, 1, _BN)

    BDn = bd(W_nb)
    BDs = bd(W_sg)
    BDp = bd(W_ped)
    BD1p = bd(W1[0:32])
    BD1n = bd(W1[32:64])
    BD1s = bd(W1[64:96])
    BD2 = bd(W2)
    BD3 = bd(W3)
    E = jnp.kron(eyeT, jnp.ones((1, 32), jnp.float32))

    tile = lambda b: jnp.tile(b, _T).reshape(1, -1)
    bn_t, bs_t, bp_t = tile(b_nb), tile(b_sg), tile(b_ped)
    b1_t, b2_t, b3_t = tile(b1), tile(b2), tile(b3)

    const = lambda shape: pl.BlockSpec(shape, lambda i: (0,) * len(shape))
    Fnb, Fsg, P, S = pl.pallas_call(
        _body,
        grid=(NB,),
        in_specs=[
            pl.BlockSpec((_BN, _T * 4), lambda i: (i, 0)),
            pl.BlockSpec((_BN, _T), lambda i: (i, 0)),
            pl.BlockSpec((1, 1, _BN), lambda i: (i, 0, 0)),
            pl.BlockSpec((_BN, _T * 2), lambda i: (i, 0)),
            pl.BlockSpec((_BN, _T), lambda i: (i, 0)),
            pl.BlockSpec((1, 1, _BN), lambda i: (i, 0, 0)),
            const((_B, _T * 4)),
            const((_T * 4, 512)), const((1, 512)),
            const((_T * 2, 512)), const((1, 512)),
            const((_T, 512)),
            const((_T * 4, 512)), const((1, 512)),
            const((512, 512)), const((512, 512)), const((512, 512)),
            const((1, 512)),
            const((512, 512)), const((1, 512)),
            const((512, _T)), const((1, _T)),
        ],
        out_specs=[
            const((_B, 512)), const((_B, 512)), const((_B, 512)),
            const((_B, _T)),
        ],
        out_shape=[
            jax.ShapeDtypeStruct((_B, 512), jnp.float32),
            jax.ShapeDtypeStruct((_B, 512), jnp.float32),
            jax.ShapeDtypeStruct((_B, 512), jnp.float32),
            jax.ShapeDtypeStruct((_B, _T), jnp.float32),
        ],
        interpret=interpret,
    )(xn2, cls_neighbor, segn3, xs2, cls_sign, segs3, xped2,
      BDn, bn_t, BDs, bs_t, E,
      BDp, bp_t, BD1p, BD1n, BD1s, b1_t, BD2, b2_t, BD3, b3_t)

    int_det_score = S.reshape(_B, _T, 1)
    all_traffic = jnp.concatenate(
        [P.reshape(_B, _T, 32), Fnb.reshape(_B, _T, 32),
         Fsg.reshape(_B, _T, 32)], axis=-1)
    return (int_det_score, all_traffic)


def kernel(x_ped, x_neighbor, x_sign, cls_neighbor, cls_sign,
           seg_neighbor, seg_sign,
           W_ped, b_ped, W_nb, b_nb, W_sg, b_sg,
           W1, b1, W2, b2, W3, b3):
    return _run(x_ped, x_neighbor, x_sign, cls_neighbor, cls_sign,
                seg_neighbor, seg_sign,
                W_ped, b_ped, W_nb, b_nb, W_sg, b_sg,
                W1, b1, W2, b2, W3, b3)


# speedup vs baseline: 5.9899x; 1.3833x over previous
import functools

import jax
import jax.numpy as jnp
from jax.experimental import pallas as pl
from jax.experimental.pallas import tpu as pltpu

_B = 16
_T = 16
_N = 32768
_BN = 2048
_NT = jnp.float32


def _dotT(a, b):
    return jax.lax.dot_general(a, b, (((1,), (1,)), ((), ())),
                               preferred_element_type=jnp.float32)


def _body(xn_ref, cn_ref, segn_ref, xs_ref, cs_ref, segs_ref, xped_ref,
          BDnT_ref, bn_ref, BDsT_ref, bs_ref, R4_ref, R2_ref,
          ErbnT_ref, ErbsT_ref,
          BDpT_ref, bp_ref, BD1pT_ref, BD1nT_ref, BD1sT_ref, b1_ref,
          BD2T_ref, b2_ref, BD3T_ref, b3_ref,
          Fnb_ref, Fsg_ref, P_ref, S_ref,
          cntn_ref, cnts_ref):
    i = pl.program_id(0)

    def agg(x_ref, c_ref, seg_ref, BDT_ref, b_ref, R_ref):
        m = (c_ref[...] != -1.0).astype(_NT)
        m4 = jnp.dot(R_ref[...], m, preferred_element_type=jnp.float32)
        xm = x_ref[...] * m4.astype(x_ref.dtype)
        em = jnp.maximum(
            jnp.dot(BDT_ref[...], xm, preferred_element_type=jnp.float32)
            + b_ref[...], 0.0)
        seg = seg_ref[...]
        oh = (jax.lax.broadcasted_iota(jnp.int32, (_B, _BN), 0)
              == jnp.broadcast_to(seg, (_B, _BN))).astype(_NT)
        f = _dotT(em.astype(_NT), oh)
        cnt = _dotT((1.0 - m), oh)
        return f, cnt

    fn, cn = agg(xn_ref, cn_ref, segn_ref, BDnT_ref, bn_ref, R4_ref)
    fs, cs = agg(xs_ref, cs_ref, segs_ref, BDsT_ref, bs_ref, R2_ref)

    @pl.when(i == 0)
    def _():
        Fnb_ref[...] = fn
        Fsg_ref[...] = fs
        cntn_ref[...] = cn
        cnts_ref[...] = cs

    @pl.when(i > 0)
    def _():
        Fnb_ref[...] += fn
        Fsg_ref[...] += fs
        cntn_ref[...] += cn
        cnts_ref[...] += cs

    @pl.when(i == pl.num_programs(0) - 1)
    def _():
        Fn = Fnb_ref[...] - jnp.dot(ErbnT_ref[...], cntn_ref[...],
                                    preferred_element_type=jnp.float32)
        Fs = Fsg_ref[...] - jnp.dot(ErbsT_ref[...], cnts_ref[...],
                                    preferred_element_type=jnp.float32)
        Fnb_ref[...] = Fn
        Fsg_ref[...] = Fs
        P = jnp.maximum(
            jnp.dot(BDpT_ref[...], xped_ref[...],
                    preferred_element_type=jnp.float32) + bp_ref[...], 0.0)
        P_ref[...] = P
        A1 = jnp.maximum(
            jnp.dot(BD1pT_ref[...], P, preferred_element_type=jnp.float32)
            + jnp.dot(BD1nT_ref[...], Fn, preferred_element_type=jnp.float32)
            + jnp.dot(BD1sT_ref[...], Fs, preferred_element_type=jnp.float32)
            + b1_ref[...], 0.0)
        A2 = jnp.maximum(
            jnp.dot(BD2T_ref[...], A1, preferred_element_type=jnp.float32)
            + b2_ref[...], 0.0)
        S_ref[...] = (jnp.dot(BD3T_ref[...], A2,
                              preferred_element_type=jnp.float32)
                      + b3_ref[...])


@functools.partial(jax.jit, static_argnames=("interpret",))
def _run(x_ped, x_neighbor, x_sign, cls_neighbor, cls_sign,
         seg_neighbor, seg_sign,
         W_ped, b_ped, W_nb, b_nb, W_sg, b_sg,
         W1, b1, W2, b2, W3, b3, interpret=False):
    NB = _N // _BN
    eyeT = jnp.eye(_T, dtype=jnp.float32)

    def bdT(W):
        return jnp.kron(eyeT, W.T)

    xnT = x_neighbor.transpose(1, 2, 0).reshape(_T * 4, _N)
    xsT = x_sign.transpose(1, 2, 0).reshape(_T * 2, _N)
    xpedT = x_ped.transpose(1, 2, 0).reshape(_T * 4, _B)
    cnT = cls_neighbor.T
    csT = cls_sign.T
    segn2 = seg_neighbor.reshape(1, _N)
    segs2 = seg_sign.reshape(1, _N)

    BDnT = bdT(W_nb)
    BDsT = bdT(W_sg)
    BDpT = bdT(W_ped)
    BD1pT = bdT(W1[0:32])
    BD1nT = bdT(W1[32:64])
    BD1sT = bdT(W1[64:96])
    BD2T = bdT(W2)
    BD3T = bdT(W3)
    R4 = jnp.kron(eyeT, jnp.ones((4, 1), jnp.float32))
    R2 = jnp.kron(eyeT, jnp.ones((2, 1), jnp.float32))
    ErbnT = jnp.kron(eyeT, jax.nn.relu(b_nb)[:, None])
    ErbsT = jnp.kron(eyeT, jax.nn.relu(b_sg)[:, None])

    col = lambda b: jnp.tile(b, _T).reshape(-1, 1)
    bn_c, bs_c, bp_c = col(b_nb), col(b_sg), col(b_ped)
    b1_c, b2_c = col(b1), col(b2)
    b3_r = jnp.tile(b3, _T).reshape(_T, 1)

    const = lambda shape: pl.BlockSpec(shape, lambda i: (0,) * len(shape))
    Fnb, Fsg, P, S = pl.pallas_call(
        _body,
        grid=(NB,),
        in_specs=[
            pl.BlockSpec((_T * 4, _BN), lambda i: (0, i)),
            pl.BlockSpec((_T, _BN), lambda i: (0, i)),
            pl.BlockSpec((1, _BN), lambda i: (0, i)),
            pl.BlockSpec((_T * 2, _BN), lambda i: (0, i)),
            pl.BlockSpec((_T, _BN), lambda i: (0, i)),
            pl.BlockSpec((1, _BN), lambda i: (0, i)),
            const((_T * 4, _B)),
            const((512, 64)), const((512, 1)),
            const((512, 32)), const((512, 1)),
            const((64, 16)), const((32, 16)),
            const((512, 16)), const((512, 16)),
            const((512, 64)), const((512, 1)),
            const((512, 512)), const((512, 512)), const((512, 512)),
            const((512, 1)),
            const((512, 512)), const((512, 1)),
            const((16, 512)), const((16, 1)),
        ],
        out_specs=[
            const((512, _B)), const((512, _B)), const((512, _B)),
            const((_T, _B)),
        ],
        out_shape=[
            jax.ShapeDtypeStruct((512, _B), jnp.float32),
            jax.ShapeDtypeStruct((512, _B), jnp.float32),
            jax.ShapeDtypeStruct((512, _B), jnp.float32),
            jax.ShapeDtypeStruct((_T, _B), jnp.float32),
        ],
        scratch_shapes=[
            pltpu.VMEM((_T, _B), jnp.float32),
            pltpu.VMEM((_T, _B), jnp.float32),
        ],
        interpret=interpret,
    )(xnT, cnT, segn2, xsT, csT, segs2, xpedT,
      BDnT, bn_c, BDsT, bs_c, R4, R2, ErbnT, ErbsT,
      BDpT, bp_c, BD1pT, BD1nT, BD1sT, b1_c, BD2T, b2_c, BD3T, b3_r)

    int_det_score = S.T.reshape(_B, _T, 1)
    all_traffic = jnp.concatenate(
        [P.T.reshape(_B, _T, 32), Fnb.T.reshape(_B, _T, 32),
         Fsg.T.reshape(_B, _T, 32)], axis=-1)
    return (int_det_score, all_traffic)


def kernel(x_ped, x_neighbor, x_sign, cls_neighbor, cls_sign,
           seg_neighbor, seg_sign,
           W_ped, b_ped, W_nb, b_nb, W_sg, b_sg,
           W1, b1, W2, b2, W3, b3):
    return _run(x_ped, x_neighbor, x_sign, cls_neighbor, cls_sign,
                seg_neighbor, seg_sign,
                W_ped, b_ped, W_nb, b_nb, W_sg, b_sg,
                W1, b1, W2, b2, W3, b3)


# speedup vs baseline: 6.8536x; 1.1442x over previous
import functools

import jax
import jax.numpy as jnp
from jax.experimental import pallas as pl
from jax.experimental.pallas import tpu as pltpu

_B = 16
_T = 16
_N = 32768
_BN = 2048
_CB = _BN // 128
_NT = jnp.float32


def _dotT(a, b):
    return jax.lax.dot_general(a, b, (((1,), (1,)), ((), ())),
                               preferred_element_type=jnp.float32)


def _body(xn_ref, cn_ref, segn_ref, xs_ref, cs_ref, segs_ref, xped_ref,
          BDnT_ref, bn_ref, BDsT_ref, bs_ref, R4_ref, R2_ref,
          ErbnT_ref, ErbsT_ref,
          BDpT_ref, bp_ref, BD1pT_ref, BD1nT_ref, BD1sT_ref, b1_ref,
          BD2T_ref, b2_ref, BD3T_ref, b3_ref,
          Fnb_ref, Fsg_ref, P_ref, S_ref,
          cntn_ref, cnts_ref):
    i = pl.program_id(0)

    def agg(x_ref, c_ref, seg_ref, BDT_ref, b_ref, R_ref):
        m = (c_ref[...] != -1.0).astype(_NT)
        m4 = jnp.dot(R_ref[...], m, preferred_element_type=jnp.float32)
        v = x_ref[...]
        d = v.shape[2]
        x = jnp.transpose(v, (0, 2, 1, 3)).reshape(_T * d, _BN)
        xm = x * m4.astype(x.dtype)
        em = jnp.maximum(
            jnp.dot(BDT_ref[...], xm, preferred_element_type=jnp.float32)
            + b_ref[...], 0.0)
        seg = seg_ref[...]
        oh = (jax.lax.broadcasted_iota(jnp.int32, (_B, _BN), 0)
              == jnp.broadcast_to(seg, (_B, _BN))).astype(_NT)
        f = _dotT(em.astype(_NT), oh)
        cnt = _dotT((1.0 - m), oh)
        return f, cnt

    fn, cn = agg(xn_ref, cn_ref, segn_ref, BDnT_ref, bn_ref, R4_ref)
    fs, cs = agg(xs_ref, cs_ref, segs_ref, BDsT_ref, bs_ref, R2_ref)

    @pl.when(i == 0)
    def _():
        Fnb_ref[...] = fn
        Fsg_ref[...] = fs
        cntn_ref[...] = cn
        cnts_ref[...] = cs

    @pl.when(i > 0)
    def _():
        Fnb_ref[...] += fn
        Fsg_ref[...] += fs
        cntn_ref[...] += cn
        cnts_ref[...] += cs

    @pl.when(i == pl.num_programs(0) - 1)
    def _():
        Fn = Fnb_ref[...] - jnp.dot(ErbnT_ref[...], cntn_ref[...],
                                    preferred_element_type=jnp.float32)
        Fs = Fsg_ref[...] - jnp.dot(ErbsT_ref[...], cnts_ref[...],
                                    preferred_element_type=jnp.float32)
        Fnb_ref[...] = Fn
        Fsg_ref[...] = Fs
        P = jnp.maximum(
            jnp.dot(BDpT_ref[...], xped_ref[...],
                    preferred_element_type=jnp.float32) + bp_ref[...], 0.0)
        P_ref[...] = P
        A1 = jnp.maximum(
            jnp.dot(BD1pT_ref[...], P, preferred_element_type=jnp.float32)
            + jnp.dot(BD1nT_ref[...], Fn, preferred_element_type=jnp.float32)
            + jnp.dot(BD1sT_ref[...], Fs, preferred_element_type=jnp.float32)
            + b1_ref[...], 0.0)
        A2 = jnp.maximum(
            jnp.dot(BD2T_ref[...], A1, preferred_element_type=jnp.float32)
            + b2_ref[...], 0.0)
        S_ref[...] = (jnp.dot(BD3T_ref[...], A2,
                              preferred_element_type=jnp.float32)
                      + b3_ref[...])


@functools.partial(jax.jit, static_argnames=("interpret",))
def _run(x_ped, x_neighbor, x_sign, cls_neighbor, cls_sign,
         seg_neighbor, seg_sign,
         W_ped, b_ped, W_nb, b_nb, W_sg, b_sg,
         W1, b1, W2, b2, W3, b3, interpret=False):
    NB = _N // _BN
    eyeT = jnp.eye(_T, dtype=jnp.float32)

    def bdT(W):
        return jnp.kron(eyeT, W.T)

    NC = _N // 128
    xn4 = (x_neighbor.transpose(1, 0, 2).reshape(_T, NC, 128, 4)
           .transpose(0, 1, 3, 2))
    xs4 = (x_sign.transpose(1, 0, 2).reshape(_T, NC, 128, 2)
           .transpose(0, 1, 3, 2))
    xpedT = x_ped.transpose(1, 2, 0).reshape(_T * 4, _B)
    cnT = cls_neighbor.T
    csT = cls_sign.T
    segn2 = seg_neighbor.reshape(1, _N)
    segs2 = seg_sign.reshape(1, _N)

    BDnT = bdT(W_nb)
    BDsT = bdT(W_sg)
    BDpT = bdT(W_ped)
    BD1pT = bdT(W1[0:32])
    BD1nT = bdT(W1[32:64])
    BD1sT = bdT(W1[64:96])
    BD2T = bdT(W2)
    BD3T = bdT(W3)
    R4 = jnp.kron(eyeT, jnp.ones((4, 1), jnp.float32))
    R2 = jnp.kron(eyeT, jnp.ones((2, 1), jnp.float32))
    ErbnT = jnp.kron(eyeT, jax.nn.relu(b_nb)[:, None])
    ErbsT = jnp.kron(eyeT, jax.nn.relu(b_sg)[:, None])

    col = lambda b: jnp.tile(b, _T).reshape(-1, 1)
    bn_c, bs_c, bp_c = col(b_nb), col(b_sg), col(b_ped)
    b1_c, b2_c = col(b1), col(b2)
    b3_r = jnp.tile(b3, _T).reshape(_T, 1)

    const = lambda shape: pl.BlockSpec(shape, lambda i: (0,) * len(shape))
    Fnb, Fsg, P, S = pl.pallas_call(
        _body,
        grid=(NB,),
        in_specs=[
            pl.BlockSpec((_T, _CB, 4, 128), lambda i: (0, i, 0, 0)),
            pl.BlockSpec((_T, _BN), lambda i: (0, i)),
            pl.BlockSpec((1, _BN), lambda i: (0, i)),
            pl.BlockSpec((_T, _CB, 2, 128), lambda i: (0, i, 0, 0)),
            pl.BlockSpec((_T, _BN), lambda i: (0, i)),
            pl.BlockSpec((1, _BN), lambda i: (0, i)),
            const((_T * 4, _B)),
            const((512, 64)), const((512, 1)),
            const((512, 32)), const((512, 1)),
            const((64, 16)), const((32, 16)),
            const((512, 16)), const((512, 16)),
            const((512, 64)), const((512, 1)),
            const((512, 512)), const((512, 512)), const((512, 512)),
            const((512, 1)),
            const((512, 512)), const((512, 1)),
            const((16, 512)), const((16, 1)),
        ],
        out_specs=[
            const((512, _B)), const((512, _B)), const((512, _B)),
            const((_T, _B)),
        ],
        out_shape=[
            jax.ShapeDtypeStruct((512, _B), jnp.float32),
            jax.ShapeDtypeStruct((512, _B), jnp.float32),
            jax.ShapeDtypeStruct((512, _B), jnp.float32),
            jax.ShapeDtypeStruct((_T, _B), jnp.float32),
        ],
        scratch_shapes=[
            pltpu.VMEM((_T, _B), jnp.float32),
            pltpu.VMEM((_T, _B), jnp.float32),
        ],
        interpret=interpret,
    )(xn4, cnT, segn2, xs4, csT, segs2, xpedT,
      BDnT, bn_c, BDsT, bs_c, R4, R2, ErbnT, ErbsT,
      BDpT, bp_c, BD1pT, BD1nT, BD1sT, b1_c, BD2T, b2_c, BD3T, b3_r)

    int_det_score = S.T.reshape(_B, _T, 1)
    all_traffic = jnp.concatenate(
        [P.T.reshape(_B, _T, 32), Fnb.T.reshape(_B, _T, 32),
         Fsg.T.reshape(_B, _T, 32)], axis=-1)
    return (int_det_score, all_traffic)


def kernel(x_ped, x_neighbor, x_sign, cls_neighbor, cls_sign,
           seg_neighbor, seg_sign,
           W_ped, b_ped, W_nb, b_nb, W_sg, b_sg,
           W1, b1, W2, b2, W3, b3):
    return _run(x_ped, x_neighbor, x_sign, cls_neighbor, cls_sign,
                seg_neighbor, seg_sign,
                W_ped, b_ped, W_nb, b_nb, W_sg, b_sg,
                W1, b1, W2, b2, W3, b3)
